# Initial kernel scaffold; baseline (speedup 1.0000x reference)
import jax
import jax.numpy as jnp
from jax.experimental import pallas as pl


def kernel(z_e_x, codebook):
    raise NotImplementedError("write your pallas kernel here")


# speedup vs baseline: 1.3946x; 1.3946x over previous
import functools

import jax
import jax.numpy as jnp
from jax.experimental import pallas as pl
from jax.experimental.pallas import tpu as pltpu


def _vq_kernel(z_ref, cb_ref, cbsq_ref, insq_ref, out_ref):
    z = z_ref[...]
    cb = cb_ref[...]
    mm = jax.lax.dot_general(
        z, cb,
        dimension_numbers=(((1,), (1,)), ((), ())),
        preferred_element_type=jnp.float32,
    )
    dist = (cbsq_ref[...] + insq_ref[...]) - 2.0 * mm
    out_ref[...] = jnp.argmin(dist, axis=1).astype(jnp.int32)[None, None, :]


def kernel(z_e_x, codebook):
    Bv, Nv, D = z_e_x.shape
    K = codebook.shape[0]
    M = Bv * Nv
    flat = z_e_x.reshape(M, D).astype(jnp.float32)
    cb = codebook.astype(jnp.float32)
    cb_sq = jnp.sum(cb * cb, axis=1).reshape(1, K)
    in_sq = jnp.sum(flat * flat, axis=1, keepdims=True)

    BM = 576
    grid = (M // BM,)
    idx = pl.pallas_call(
        _vq_kernel,
        grid=grid,
        in_specs=[
            pl.BlockSpec((BM, D), lambda i: (i, 0)),
            pl.BlockSpec((K, D), lambda i: (0, 0)),
            pl.BlockSpec((1, K), lambda i: (0, 0)),
            pl.BlockSpec((BM, 1), lambda i: (i, 0)),
        ],
        out_specs=pl.BlockSpec((1, 1, BM), lambda i: (i, 0, 0)),
        out_shape=jax.ShapeDtypeStruct((M // BM, 1, BM), jnp.int32),
    )(flat, cb, cb_sq, in_sq)
    return idx.reshape(Bv, Nv)


# speedup vs baseline: 1.5859x; 1.1371x over previous
import functools

import jax
import jax.numpy as jnp
from jax.experimental import pallas as pl
from jax.experimental.pallas import tpu as pltpu


def _vq_kernel(z_ref, cb_ref, cbsq_ref, insq_ref, out_ref):
    z = z_ref[...]
    cb = cb_ref[...]
    mm = jax.lax.dot_general(
        z, cb,
        dimension_numbers=(((1,), (1,)), ((), ())),
        preferred_element_type=jnp.float32,
    )
    dist = (cbsq_ref[...] + insq_ref[...]) + mm
    out_ref[...] = jnp.argmin(dist, axis=1).astype(jnp.int32)[None, None, :]


def kernel(z_e_x, codebook):
    Bv, Nv, D = z_e_x.shape
    K = codebook.shape[0]
    M = Bv * Nv
    flat = z_e_x.reshape(M, D).astype(jnp.float32)
    cb = codebook.astype(jnp.float32)
    cb_sq = jnp.sum(cb * cb, axis=1).reshape(1, K)
    in_sq = jnp.sum(flat * flat, axis=1, keepdims=True)
    flat_m2 = flat * (-2.0)

    BM = 576
    grid = (M // BM,)
    idx = pl.pallas_call(
        _vq_kernel,
        grid=grid,
        in_specs=[
            pl.BlockSpec((BM, D), lambda i: (i, 0)),
            pl.BlockSpec((K, D), lambda i: (0, 0)),
            pl.BlockSpec((1, K), lambda i: (0, 0)),
            pl.BlockSpec((BM, 1), lambda i: (i, 0)),
        ],
        out_specs=pl.BlockSpec((1, 1, BM), lambda i: (i, 0, 0)),
        out_shape=jax.ShapeDtypeStruct((M // BM, 1, BM), jnp.int32),
    )(flat_m2, cb, cb_sq, in_sq)
    return idx.reshape(Bv, Nv)
